# Initial kernel scaffold; baseline (speedup 1.0000x reference)
import jax
import jax.numpy as jnp
from jax.experimental import pallas as pl


def kernel(x, pos, W_so, b_so, W_aw, b_aw, W_v, b_v, W_o, b_o, shepard_power):
    raise NotImplementedError("write your pallas kernel here")


# speedup vs baseline: 35.4064x; 35.4064x over previous
import jax
import jax.numpy as jnp
from jax.experimental import pallas as pl
from jax.experimental.pallas import tpu as pltpu

H = 16
K = 4
NT = 256
QT = 512


def _proj_body(x_ref, pos_ref, wso_ref, bso_ref, waw_ref, baw_ref,
               wv_ref, bv_ref, samp_ref, attn_ref, val_ref):
    x = x_ref[0]
    so = jnp.dot(x, wso_ref[...], preferred_element_type=jnp.float32) + bso_ref[...]
    aw = jnp.dot(x, waw_ref[...], preferred_element_type=jnp.float32) + baw_ref[...]
    vv = jnp.dot(x, wv_ref[...], preferred_element_type=jnp.float32) + bv_ref[...]
    samp_ref[0] = so + jnp.tile(pos_ref[0], (1, so.shape[1] // 2))
    hk = aw.shape[1]
    gi = jax.lax.broadcasted_iota(jnp.int32, (hk, hk), 0)
    gj = jax.lax.broadcasted_iota(jnp.int32, (hk, hk), 1)
    g = ((gi // K) == (gj // K)).astype(jnp.float32)
    e = jnp.exp(aw - jnp.max(aw, axis=1, keepdims=True))
    attn_ref[0] = e / jnp.dot(e, g, preferred_element_type=jnp.float32)
    val_ref[0] = vv


def _knn_body(pw_ref, s_ref, p_ref, v_ref, out_ref):
    pwr = jnp.maximum(pw_ref[0, 0], 0.0) + 1e-6
    s = s_ref[0]
    p = p_ref[0]
    sp = jnp.dot(s[:, 0:2], p[0:2, :], preferred_element_type=jnp.float32)
    s2 = s[:, 0:1] * s[:, 0:1] + s[:, 1:2] * s[:, 1:2]
    d2 = (s2 + p[2:3, :]) - 2.0 * sp
    dm = d2
    for _ in range(3):
        mi = jnp.min(dm, axis=1, keepdims=True)
        dm = jnp.where(dm <= mi, jnp.inf, dm)
    t4 = jnp.min(dm, axis=1, keepdims=True)
    dx = s[:, 0:1] - p[0:1, :]
    dy = s[:, 1:2] - p[1:2, :]
    dist = jnp.sqrt(dx * dx + dy * dy) + 1e-6
    w = jnp.where(d2 <= t4, jnp.exp(-pwr * dist), 0.0)
    w = w * (s[:, 2:3] / jnp.sum(w, axis=1, keepdims=True))
    outq = jnp.dot(w, v_ref[0], preferred_element_type=jnp.float32)
    ri = jax.lax.broadcasted_iota(jnp.int32, (QT // K, QT), 0)
    ci = jax.lax.broadcasted_iota(jnp.int32, (QT // K, QT), 1)
    r = ((ci // K) == ri).astype(jnp.float32)
    out_ref[0] = jnp.dot(r, outq, preferred_element_type=jnp.float32)


def _out_body(x_ref, w_ref, b_ref, o_ref):
    o_ref[...] = (jnp.dot(x_ref[...], w_ref[...],
                          preferred_element_type=jnp.float32) + b_ref[...])


def kernel(x, pos, W_so, b_so, W_aw, b_aw, W_v, b_v, W_o, b_o, shepard_power):
    b, n, d = x.shape
    c = d // H
    bh, nk = b * H, n * K

    samp, attn, vals = pl.pallas_call(
        _proj_body,
        grid=(b, n // NT),
        in_specs=[
            pl.BlockSpec((1, NT, d), lambda i, j: (i, j, 0)),
            pl.BlockSpec((1, NT, 2), lambda i, j: (i, j, 0)),
            pl.BlockSpec((d, H * K * 2), lambda i, j: (0, 0)),
            pl.BlockSpec((1, H * K * 2), lambda i, j: (0, 0)),
            pl.BlockSpec((d, H * K), lambda i, j: (0, 0)),
            pl.BlockSpec((1, H * K), lambda i, j: (0, 0)),
            pl.BlockSpec((d, d), lambda i, j: (0, 0)),
            pl.BlockSpec((1, d), lambda i, j: (0, 0)),
        ],
        out_specs=[
            pl.BlockSpec((1, NT, H * K * 2), lambda i, j: (i, j, 0)),
            pl.BlockSpec((1, NT, H * K), lambda i, j: (i, j, 0)),
            pl.BlockSpec((1, NT, d), lambda i, j: (i, j, 0)),
        ],
        out_shape=[
            jax.ShapeDtypeStruct((b, n, H * K * 2), jnp.float32),
            jax.ShapeDtypeStruct((b, n, H * K), jnp.float32),
            jax.ShapeDtypeStruct((b, n, d), jnp.float32),
        ],
    )(x, pos, W_so.T, b_so[None, :], W_aw.T, b_aw[None, :],
      W_v.T, b_v[None, :])

    samp_q = samp.reshape(b, n, H, K, 2).transpose(0, 2, 1, 3, 4).reshape(bh, nk, 2)
    attn_q = attn.reshape(b, n, H, K).transpose(0, 2, 1, 3).reshape(bh, nk, 1)
    sblk = jnp.concatenate([samp_q, attn_q], axis=-1)
    vals_h = vals.reshape(b, n, H, c).transpose(0, 2, 1, 3).reshape(bh, n, c)
    px, py = pos[..., 0], pos[..., 1]
    paug = jnp.stack([px, py, px * px + py * py], axis=1)

    out_h = pl.pallas_call(
        _knn_body,
        grid=(bh, nk // QT),
        in_specs=[
            pl.BlockSpec(memory_space=pltpu.SMEM),
            pl.BlockSpec((1, QT, 3), lambda i, j: (i, j, 0)),
            pl.BlockSpec((1, 3, n), lambda i, j: (i // H, 0, 0)),
            pl.BlockSpec((1, n, c), lambda i, j: (i, 0, 0)),
        ],
        out_specs=pl.BlockSpec((1, QT // K, c), lambda i, j: (i, j, 0)),
        out_shape=jax.ShapeDtypeStruct((bh, n, c), jnp.float32),
    )(shepard_power.reshape(1, 1), sblk, paug, vals_h)

    y = out_h.reshape(b, H, n, c).transpose(0, 2, 1, 3).reshape(b * n, d)
    out = pl.pallas_call(
        _out_body,
        grid=(b * n // NT,),
        in_specs=[
            pl.BlockSpec((NT, d), lambda i: (i, 0)),
            pl.BlockSpec((d, d), lambda i: (0, 0)),
            pl.BlockSpec((1, d), lambda i: (0, 0)),
        ],
        out_specs=pl.BlockSpec((NT, d), lambda i: (i, 0)),
        out_shape=jax.ShapeDtypeStruct((b * n, d), jnp.float32),
    )(y, W_o.T, b_o[None, :])
    return out.reshape(b, n, d)


# speedup vs baseline: 45.7118x; 1.2911x over previous
import jax
import jax.numpy as jnp
from jax.experimental import pallas as pl
from jax.experimental.pallas import tpu as pltpu

H = 16
K = 4
NT = 256
QT = 512


def _proj_body(x_ref, pos_ref, wso_ref, bso_ref, waw_ref, baw_ref,
               wv_ref, bv_ref, samp_ref, attn_ref, val_ref):
    x = x_ref[0]
    so = jnp.dot(x, wso_ref[...], preferred_element_type=jnp.float32) + bso_ref[...]
    aw = jnp.dot(x, waw_ref[...], preferred_element_type=jnp.float32) + baw_ref[...]
    vv = jnp.dot(x, wv_ref[...], preferred_element_type=jnp.float32) + bv_ref[...]
    samp_ref[0] = so + jnp.tile(pos_ref[0], (1, so.shape[1] // 2))
    hk = aw.shape[1]
    gi = jax.lax.broadcasted_iota(jnp.int32, (hk, hk), 0)
    gj = jax.lax.broadcasted_iota(jnp.int32, (hk, hk), 1)
    g = ((gi // K) == (gj // K)).astype(jnp.float32)
    e = jnp.exp(aw - jnp.max(aw, axis=1, keepdims=True))
    attn_ref[0] = e / jnp.dot(e, g, preferred_element_type=jnp.float32)
    val_ref[0] = vv


def _knn_body(pw_ref, s_ref, a_ref, p_ref, v_ref, out_ref):
    k = pl.program_id(1)
    pwr = jnp.maximum(pw_ref[0, 0], 0.0) + 1e-6
    s = s_ref[0, 0]
    p = p_ref[0]
    sx, sy = s[:, 0:1], s[:, 1:2]
    sp = jnp.dot(s, p[0:2, :], preferred_element_type=jnp.float32)
    d2 = ((sx * sx + sy * sy) + p[2:3, :]) - 2.0 * sp
    dm = d2
    for _ in range(3):
        mi = jnp.min(dm, axis=1, keepdims=True)
        dm = jnp.where(dm <= mi, jnp.inf, dm)
    t4 = jnp.min(dm, axis=1, keepdims=True)
    dx = sx - p[0:1, :]
    dy = sy - p[1:2, :]
    dist = jnp.sqrt(dx * dx + dy * dy) + 1e-6
    w = jnp.where(d2 <= t4, jnp.exp(-pwr * dist), 0.0)
    a = a_ref[0, 0]
    lane = jax.lax.broadcasted_iota(jnp.int32, a.shape, 1)
    attn_col = jnp.sum(jnp.where(lane == k, a, 0.0), axis=1, keepdims=True)
    w = w * (attn_col / jnp.sum(w, axis=1, keepdims=True))
    part = jnp.dot(w, v_ref[0, 0], preferred_element_type=jnp.float32)

    @pl.when(k == 0)
    def _():
        out_ref[0, 0] = part

    @pl.when(k > 0)
    def _():
        out_ref[0, 0] += part


def _out_body(x_ref, w_ref, b_ref, o_ref):
    j = pl.program_id(1)
    part = jnp.dot(x_ref[0, 0], w_ref[0], preferred_element_type=jnp.float32)

    @pl.when(j == 0)
    def _():
        o_ref[0] = part + b_ref[...]

    @pl.when(j > 0)
    def _():
        o_ref[0] += part


def kernel(x, pos, W_so, b_so, W_aw, b_aw, W_v, b_v, W_o, b_o, shepard_power):
    b, n, d = x.shape
    c = d // H
    bh, nk = b * H, n * K

    samp, attn, vals = pl.pallas_call(
        _proj_body,
        grid=(b, n // NT),
        in_specs=[
            pl.BlockSpec((1, NT, d), lambda i, j: (i, j, 0)),
            pl.BlockSpec((1, NT, 2), lambda i, j: (i, j, 0)),
            pl.BlockSpec((d, H * K * 2), lambda i, j: (0, 0)),
            pl.BlockSpec((1, H * K * 2), lambda i, j: (0, 0)),
            pl.BlockSpec((d, H * K), lambda i, j: (0, 0)),
            pl.BlockSpec((1, H * K), lambda i, j: (0, 0)),
            pl.BlockSpec((d, d), lambda i, j: (0, 0)),
            pl.BlockSpec((1, d), lambda i, j: (0, 0)),
        ],
        out_specs=[
            pl.BlockSpec((1, NT, H * K * 2), lambda i, j: (i, j, 0)),
            pl.BlockSpec((1, NT, H * K), lambda i, j: (i, j, 0)),
            pl.BlockSpec((1, NT, d), lambda i, j: (i, j, 0)),
        ],
        out_shape=[
            jax.ShapeDtypeStruct((b, n, H * K * 2), jnp.float32),
            jax.ShapeDtypeStruct((b, n, H * K), jnp.float32),
            jax.ShapeDtypeStruct((b, n, d), jnp.float32),
        ],
    )(x, pos, W_so.T, b_so[None, :], W_aw.T, b_aw[None, :],
      W_v.T, b_v[None, :])

    px, py = pos[..., 0], pos[..., 1]
    paug = jnp.stack([px, py, px * px + py * py], axis=1)
    samp4 = samp.reshape(b, n, H * K, 2).transpose(0, 2, 1, 3)
    attn4 = attn.reshape(b, n, H, K).transpose(0, 2, 1, 3)
    vals4 = vals.reshape(b, n, H, c).transpose(0, 2, 1, 3)

    out_h = pl.pallas_call(
        _knn_body,
        grid=(bh, K),
        in_specs=[
            pl.BlockSpec(memory_space=pltpu.SMEM),
            pl.BlockSpec((1, 1, n, 2), lambda i, j: (i // H, (i % H) * K + j, 0, 0)),
            pl.BlockSpec((1, 1, n, K), lambda i, j: (i // H, i % H, 0, 0)),
            pl.BlockSpec((1, 3, n), lambda i, j: (i // H, 0, 0)),
            pl.BlockSpec((1, 1, n, c), lambda i, j: (i // H, i % H, 0, 0)),
        ],
        out_specs=pl.BlockSpec((1, 1, n, c), lambda i, j: (i // H, i % H, 0, 0)),
        out_shape=jax.ShapeDtypeStruct((b, H, n, c), jnp.float32),
    )(shepard_power.reshape(1, 1), samp4, attn4, paug, vals4)

    out = pl.pallas_call(
        _out_body,
        grid=(b, H),
        in_specs=[
            pl.BlockSpec((1, 1, n, c), lambda i, j: (i, j, 0, 0)),
            pl.BlockSpec((1, c, d), lambda i, j: (j, 0, 0)),
            pl.BlockSpec((1, d), lambda i, j: (0, 0)),
        ],
        out_specs=pl.BlockSpec((1, n, d), lambda i, j: (i, 0, 0)),
        out_shape=jax.ShapeDtypeStruct((b, n, d), jnp.float32),
    )(out_h, W_o.T.reshape(H, c, d), b_o[None, :])
    return out
